# baseline (speedup 1.0000x reference)
import functools

import jax
import jax.numpy as jnp
from jax import lax
from jax.experimental import pallas as pl
from jax.experimental.pallas import tpu as pltpu
from jax.experimental.pallas import tpu_sc as plsc

_VOCAB = 1000000
_EMB = 64
_B = 4096
_L = 200

_NC = 2
_NS = 16
_NW = _NC * _NS
_BPW = _B // _NW
_LANES = 16

_CH = 10000
_NCH = _VOCAB // _CH
_RING = 4
_NOUT = _NCH // _RING


def _fold_body(tab_hbm, w_ref, b_ref, out_hbm, b0, b1, b2, b3, tv, sems, osem):
    bufs = (b0, b1, b2, b3)
    w = w_ref[...]
    bias = b_ref[0, 0]

    for r in range(_RING):
        pltpu.make_async_copy(
            tab_hbm.at[pl.ds(r * _CH, _CH), :], bufs[r], sems.at[r]
        ).start()

    def outer(g, carry):
        for r in range(_RING):
            c = g * _RING + r
            pltpu.make_async_copy(
                tab_hbm.at[pl.ds(c * _CH, _CH), :], bufs[r], sems.at[r]
            ).wait()
            x = bufs[r][...]
            y = lax.dot_general(
                w, x, (((1,), (1,)), ((), ())),
                preferred_element_type=jnp.float32,
                precision=lax.Precision.DEFAULT,
            ) + bias
            tv[pl.ds(c, 1), :] = y

            @pl.when(c + _RING < _NCH)
            def _():
                pltpu.make_async_copy(
                    tab_hbm.at[pl.ds((c + _RING) * _CH, _CH), :],
                    bufs[r],
                    sems.at[r],
                ).start()

        return carry

    lax.fori_loop(0, _NOUT, outer, 0)
    pltpu.async_copy(tv, out_hbm, osem).wait()


def _fold_table(table, w_scaled, b_scaled):
    return pl.pallas_call(
        _fold_body,
        in_specs=[
            pl.BlockSpec(memory_space=pl.ANY),
            pl.BlockSpec(memory_space=pltpu.VMEM),
            pl.BlockSpec(memory_space=pltpu.VMEM),
        ],
        out_specs=pl.BlockSpec(memory_space=pl.ANY),
        out_shape=jax.ShapeDtypeStruct((_NCH, _CH), jnp.float32),
        scratch_shapes=[
            pltpu.VMEM((_CH, _EMB), jnp.float32),
            pltpu.VMEM((_CH, _EMB), jnp.float32),
            pltpu.VMEM((_CH, _EMB), jnp.float32),
            pltpu.VMEM((_CH, _EMB), jnp.float32),
            pltpu.VMEM((_NCH, _CH), jnp.float32),
            pltpu.SemaphoreType.DMA((_RING,)),
            pltpu.SemaphoreType.DMA,
        ],
    )(table, w_scaled, b_scaled)


_mesh = plsc.VectorSubcoreMesh(core_axis_name="c", subcore_axis_name="s")

_GRP = 8


@functools.partial(
    pl.kernel,
    mesh=_mesh,
    out_type=jax.ShapeDtypeStruct((_B,), jnp.float32),
    scratch_types=[
        pltpu.VMEM((_L, _BPW), jnp.int32),
        pltpu.VMEM((_L, _BPW), jnp.float32),
        pltpu.VMEM((_BPW,), jnp.float32),
        pltpu.SemaphoreType.DMA,
    ],
)
def _pool_kernel(t_hbm, idx_hbm, out_hbm, idx_v, vals_v, res_v, sem):
    wid = lax.axis_index("s") * _NC + lax.axis_index("c")

    pltpu.sync_copy(idx_hbm.at[wid], idx_v)

    @pl.loop(0, _L, step=_GRP)
    def _gather(l0):
        for j in range(_GRP):
            pltpu.async_copy(
                t_hbm.at[idx_v.at[l0 + j]], vals_v.at[l0 + j], sem
            )
        for j in range(_GRP):
            pltpu.make_async_copy(
                t_hbm.at[idx_v.at[l0 + j]], vals_v.at[l0 + j], sem
            ).wait()

    def _acc(c, accs):
        return tuple(
            accs[j] + vals_v[c, pl.ds(j * _LANES, _LANES)] for j in range(8)
        )

    accs = lax.fori_loop(
        0, _L, _acc, tuple(jnp.zeros((_LANES,), jnp.float32) for _ in range(8))
    )

    for j in range(8):
        y = accs[j]
        res_v[pl.ds(j * _LANES, _LANES)] = 1.0 / (1.0 + jnp.exp(-y))

    pltpu.sync_copy(res_v, out_hbm.at[pl.ds(wid * _BPW, _BPW)])


def kernel(ids, table, W, b):
    w_scaled = (W * (1.0 / _L)).astype(jnp.float32)
    b_scaled = (b * (1.0 / _L)).reshape(1, 1).astype(jnp.float32)
    t = _fold_table(table, w_scaled, b_scaled).reshape(_VOCAB)
    idsp = ids.reshape(_NW, _BPW, _L).transpose(0, 2, 1)
    out = _pool_kernel(t, idsp)
    return out.reshape(_B, 1)


# speedup vs baseline: 1.0166x; 1.0166x over previous
import functools

import jax
import jax.numpy as jnp
from jax import lax
from jax.experimental import pallas as pl
from jax.experimental.pallas import tpu as pltpu
from jax.experimental.pallas import tpu_sc as plsc

_VOCAB = 1000000
_EMB = 64
_B = 4096
_L = 200

_NC = 2
_NS = 16
_NW = _NC * _NS
_BPW = _B // _NW
_LANES = 16

_CH = 10000
_NCH = _VOCAB // _CH
_RING = 4
_NOUT = _NCH // _RING


def _fold_body(tab_hbm, w_ref, b_ref, out_hbm, b0, b1, b2, b3, tv, sems, osem):
    bufs = (b0, b1, b2, b3)
    w = w_ref[...]
    bias = b_ref[0, 0]

    for r in range(_RING):
        pltpu.make_async_copy(
            tab_hbm.at[pl.ds(r * _CH, _CH), :], bufs[r], sems.at[r]
        ).start()

    def outer(g, carry):
        for r in range(_RING):
            c = g * _RING + r
            pltpu.make_async_copy(
                tab_hbm.at[pl.ds(c * _CH, _CH), :], bufs[r], sems.at[r]
            ).wait()
            x = bufs[r][...]
            y = lax.dot_general(
                w, x, (((1,), (1,)), ((), ())),
                preferred_element_type=jnp.float32,
                precision=lax.Precision.DEFAULT,
            ) + bias
            tv[pl.ds(c, 1), :] = y

            @pl.when(c + _RING < _NCH)
            def _():
                pltpu.make_async_copy(
                    tab_hbm.at[pl.ds((c + _RING) * _CH, _CH), :],
                    bufs[r],
                    sems.at[r],
                ).start()

        return carry

    lax.fori_loop(0, _NOUT, outer, 0)
    pltpu.async_copy(tv, out_hbm, osem).wait()


def _fold_table(table, w_scaled, b_scaled):
    return pl.pallas_call(
        _fold_body,
        in_specs=[
            pl.BlockSpec(memory_space=pl.ANY),
            pl.BlockSpec(memory_space=pltpu.VMEM),
            pl.BlockSpec(memory_space=pltpu.VMEM),
        ],
        out_specs=pl.BlockSpec(memory_space=pl.ANY),
        out_shape=jax.ShapeDtypeStruct((_NCH, _CH), jnp.float32),
        scratch_shapes=[
            pltpu.VMEM((_CH, _EMB), jnp.float32),
            pltpu.VMEM((_CH, _EMB), jnp.float32),
            pltpu.VMEM((_CH, _EMB), jnp.float32),
            pltpu.VMEM((_CH, _EMB), jnp.float32),
            pltpu.VMEM((_NCH, _CH), jnp.float32),
            pltpu.SemaphoreType.DMA((_RING,)),
            pltpu.SemaphoreType.DMA,
        ],
    )(table, w_scaled, b_scaled)


_mesh = plsc.VectorSubcoreMesh(core_axis_name="c", subcore_axis_name="s")

_GRP = 16


@functools.partial(
    pl.kernel,
    mesh=_mesh,
    out_type=jax.ShapeDtypeStruct((_B,), jnp.float32),
    scratch_types=[
        pltpu.VMEM((_L, _BPW), jnp.int32),
        pltpu.VMEM((_L, _BPW), jnp.float32),
        pltpu.VMEM((_BPW,), jnp.float32),
        pltpu.SemaphoreType.DMA,
    ],
)
def _pool_kernel(t_hbm, idx_hbm, out_hbm, idx_v, vals_v, res_v, sem):
    wid = lax.axis_index("s") * _NC + lax.axis_index("c")

    pltpu.sync_copy(idx_hbm.at[wid], idx_v)

    @pl.loop(0, _L, step=_GRP)
    def _gather(l0):
        for j in range(_GRP):
            pltpu.async_copy(
                t_hbm.at[idx_v.at[l0 + j]], vals_v.at[l0 + j], sem
            )
        for j in range(_GRP):
            pltpu.make_async_copy(
                t_hbm.at[idx_v.at[l0 + j]], vals_v.at[l0 + j], sem
            ).wait()

    def _acc(c, accs):
        return tuple(
            accs[j] + vals_v[c, pl.ds(j * _LANES, _LANES)] for j in range(8)
        )

    accs = lax.fori_loop(
        0, _L, _acc, tuple(jnp.zeros((_LANES,), jnp.float32) for _ in range(8))
    )

    for j in range(8):
        y = accs[j]
        res_v[pl.ds(j * _LANES, _LANES)] = 1.0 / (1.0 + jnp.exp(-y))

    pltpu.sync_copy(res_v, out_hbm.at[pl.ds(wid * _BPW, _BPW)])


def kernel(ids, table, W, b):
    w_scaled = (W * (1.0 / _L)).astype(jnp.float32)
    b_scaled = (b * (1.0 / _L)).reshape(1, 1).astype(jnp.float32)
    t = _fold_table(table, w_scaled, b_scaled).reshape(_VOCAB)
    idsp = ids.reshape(_NW, _BPW, _L).transpose(0, 2, 1)
    out = _pool_kernel(t, idsp)
    return out.reshape(_B, 1)
